# baseline (speedup 1.0000x reference)
import jax
import jax.numpy as jnp
from jax.experimental import pallas as pl

DEG = 32
D = 128
BLK = 400


def _body(src_ref, na_ref, nb_ref, w_ref, b_ref, out_ref):
    aggr = (jnp.sum(na_ref[...], axis=1) + jnp.sum(nb_ref[...], axis=1)) * (1.0 / DEG)
    h = jnp.dot(aggr, w_ref[...], preferred_element_type=jnp.float32)
    h = h + jnp.dot(src_ref[...], b_ref[...], preferred_element_type=jnp.float32)
    out_ref[...] = jnp.maximum(h, 0.0)


def kernel(src_node_features, neighbor_node_features, W_agg, b):
    n = src_node_features.shape[0]
    grid = (n // BLK,)
    return pl.pallas_call(
        _body,
        grid=grid,
        in_specs=[
            pl.BlockSpec((BLK, D), lambda i: (i, 0)),
            pl.BlockSpec((BLK, DEG // 2, D), lambda i: (i, 0, 0)),
            pl.BlockSpec((BLK, DEG // 2, D), lambda i: (i, 1, 0)),
            pl.BlockSpec((D, D), lambda i: (0, 0)),
            pl.BlockSpec((D, D), lambda i: (0, 0)),
        ],
        out_specs=pl.BlockSpec((BLK, D), lambda i: (i, 0)),
        out_shape=jax.ShapeDtypeStruct((n, D), jnp.float32),
    )(src_node_features, neighbor_node_features, neighbor_node_features,
      W_agg, b)


# speedup vs baseline: 1.0315x; 1.0315x over previous
import jax
import jax.numpy as jnp
from jax.experimental import pallas as pl

DEG = 32
D = 128
BLK = 400
HALF = BLK // 2


def _body(src_ref, na_ref, nb_ref, w_ref, b_ref, out_ref):
    sa = jnp.sum(na_ref[...], axis=1)
    sb = jnp.sum(nb_ref[...], axis=1)
    aggr = jnp.concatenate([sa, sb], axis=0) * (1.0 / DEG)
    h = jnp.dot(aggr, w_ref[...], preferred_element_type=jnp.float32)
    h = h + jnp.dot(src_ref[...], b_ref[...], preferred_element_type=jnp.float32)
    out_ref[...] = jnp.maximum(h, 0.0)


def kernel(src_node_features, neighbor_node_features, W_agg, b):
    n = src_node_features.shape[0]
    grid = (n // BLK,)
    return pl.pallas_call(
        _body,
        grid=grid,
        in_specs=[
            pl.BlockSpec((BLK, D), lambda i: (i, 0)),
            pl.BlockSpec((HALF, DEG, D), lambda i: (2 * i, 0, 0)),
            pl.BlockSpec((HALF, DEG, D), lambda i: (2 * i + 1, 0, 0)),
            pl.BlockSpec((D, D), lambda i: (0, 0)),
            pl.BlockSpec((D, D), lambda i: (0, 0)),
        ],
        out_specs=pl.BlockSpec((BLK, D), lambda i: (i, 0)),
        out_shape=jax.ShapeDtypeStruct((n, D), jnp.float32),
    )(src_node_features, neighbor_node_features, neighbor_node_features,
      W_agg, b)
